# baseline (speedup 1.0000x reference)
import jax
import jax.numpy as jnp
from jax import lax
from jax.experimental import pallas as pl
from jax.experimental.pallas import tpu as pltpu
from jax.experimental.pallas import tpu_sc as plsc

_BM = 1024
_NUM_CORES = 2
_NUM_SUBCORES = 16
_LANES = 16


def _proj_body(h_ref, w_ref, bt_ref, logits_ref):
    h = h_ref[...]
    w = w_ref[...]
    bt = bt_ref[...]
    logits_ref[...] = lax.dot_general(
        w, h, (((0,), (1,)), ((), ())),
        preferred_element_type=jnp.float32) + bt


def _route_body(logits_hbm, probs_hbm, depth_hbm, lbuf, pbuf, dbuf, sem):
    c = lax.axis_index("c")
    s = lax.axis_index("s")
    n_tec = _NUM_CORES * _NUM_SUBCORES
    C, N = logits_hbm.shape
    tpt = N // n_tec
    t0 = (c * _NUM_SUBCORES + s) * tpt

    cp_in = pltpu.make_async_copy(
        logits_hbm.at[:, pl.ds(t0, tpt)], lbuf, sem)
    cp_in.start()
    cp_in.wait()

    def group(g, carry):
        base = g * _LANES
        l = [lbuf[j, pl.ds(base, _LANES)] for j in range(C)]
        m = l[0]
        for j in range(1, C):
            m = jnp.maximum(m, l[j])
        e = [jnp.exp(x - m) for x in l]
        ssum = e[0]
        for j in range(1, C):
            ssum = ssum + e[j]
        for j in range(C):
            pbuf[j, pl.ds(base, _LANES)] = e[j] / ssum
        idx = jnp.full((_LANES,), C - 1, dtype=jnp.int32)
        for j in range(C - 2, -1, -1):
            idx = jnp.where(l[j] == m, jnp.full((_LANES,), j, jnp.int32), idx)
        dbuf[pl.ds(base, _LANES)] = idx + 1
        return carry

    lax.fori_loop(0, tpt // _LANES, group, 0)

    cp_p = pltpu.make_async_copy(pbuf, probs_hbm.at[:, pl.ds(t0, tpt)], sem)
    cp_p.start()
    cp_p.wait()
    cp_d = pltpu.make_async_copy(dbuf, depth_hbm.at[0, pl.ds(t0, tpt)], sem)
    cp_d.start()
    cp_d.wait()


def kernel(hidden_states, W, b):
    B, S, D = hidden_states.shape
    C = W.shape[-1]
    N = B * S
    h2 = hidden_states.reshape(N, D)
    bt = b.reshape(C, 1)

    logitsT = pl.pallas_call(
        _proj_body,
        grid=(N // _BM,),
        in_specs=[
            pl.BlockSpec((_BM, D), lambda i: (i, 0)),
            pl.BlockSpec((D, C), lambda i: (0, 0)),
            pl.BlockSpec((C, 1), lambda i: (0, 0)),
        ],
        out_specs=pl.BlockSpec((C, _BM), lambda i: (0, i)),
        out_shape=jax.ShapeDtypeStruct((C, N), jnp.float32),
        compiler_params=pltpu.CompilerParams(
            dimension_semantics=("arbitrary",),
        ),
    )(h2, W, bt)

    tpt = N // (_NUM_CORES * _NUM_SUBCORES)
    probsT, depthT = pl.kernel(
        _route_body,
        out_type=[
            jax.ShapeDtypeStruct((C, N), jnp.float32),
            jax.ShapeDtypeStruct((1, N), jnp.int32),
        ],
        mesh=plsc.VectorSubcoreMesh(
            core_axis_name="c", subcore_axis_name="s",
            num_cores=_NUM_CORES, num_subcores=_NUM_SUBCORES),
        scratch_types=[
            pltpu.VMEM((C, tpt), jnp.float32),
            pltpu.VMEM((C, tpt), jnp.float32),
            pltpu.VMEM((tpt,), jnp.int32),
            pltpu.SemaphoreType.DMA,
        ],
    )(logitsT)

    depth_values = depthT.reshape(B, S)
    last_loss = jnp.zeros((), dtype=jnp.float32)
    return (depth_values, probsT.T.reshape(B, S, C),
            logitsT.T.reshape(B, S, C), last_loss)
